# Initial kernel scaffold; baseline (speedup 1.0000x reference)
import jax
import jax.numpy as jnp
from jax.experimental import pallas as pl


def kernel(predict_76, predict_38, predict_19, label, pos_thresh, neg_thresh):
    raise NotImplementedError("write your pallas kernel here")


# speedup vs baseline: 72.0991x; 72.0991x over previous
import functools

import jax
import jax.numpy as jnp
from jax.experimental import pallas as pl

_ANCH = {76: ((28.0, 28.0), (46.0, 45.0), (64.0, 66.0)),
         38: ((102.0, 74.0), (78.0, 115.0), (132.0, 113.0)),
         19: ((149.0, 163.0), (174.0, 268.0), (257.0, 176.0))}

_NBOX = 50


def _slog(x):
    return jnp.maximum(jnp.log(x), -100.0)


def _scale_kernel(raw_ref, lab_ref, pt_ref, nt_ref, out_ref, *, size):
    ss = size * size
    stride = 608 // size
    anchors = _ANCH[size]
    lab = lab_ref[...]
    pt = pt_ref[0, 0]
    nt = nt_ref[0, 0]

    cls = lab[:, 0:1]
    bx = lab[:, 1:2]
    by = lab[:, 2:3]
    bw = lab[:, 3:4]
    bh = lab[:, 4:5]

    rowsum = jnp.sum(lab, axis=1, keepdims=True)
    n = jnp.sum(jnp.where(rowsum > 0.0, 1.0, 0.0))
    tvec = jax.lax.broadcasted_iota(jnp.int32, (_NBOX, 1), 0).astype(jnp.float32)
    validf = jnp.where(tvec < n, 1.0, 0.0)
    has_box = jnp.where(n > 0.0, 1.0, 0.0)

    cell = jnp.floor(bx / stride) * size + jnp.floor(cls / stride)

    sio = jax.lax.broadcasted_iota(jnp.int32, (1, ss), 1).astype(jnp.float32)
    gy = jnp.floor(sio / size)
    gx = sio - gy * size

    hit_cell = jnp.where(sio == cell, 1.0, 0.0)

    area_b = bw * bh
    bx1 = bx - bw * 0.5
    bx2 = bx + bw * 0.5
    by1 = by - bh * 0.5
    by2 = by + bh * 0.5

    ciota = jax.lax.broadcasted_iota(jnp.int32, (80, 1), 0).astype(jnp.float32)
    onehot = jnp.where(ciota == jnp.transpose(cls), 1.0, 0.0)

    s_pos = jnp.float32(0.0)
    s_neg = jnp.float32(0.0)
    s_box = jnp.float32(0.0)
    s_bp = jnp.float32(0.0)
    s_bn = jnp.float32(0.0)
    s_cls = jnp.float32(0.0)

    for a in range(3):
        aw, ah = anchors[a]
        base = a * 85

        inter = jnp.minimum(bw, aw) * jnp.minimum(bh, ah)
        piou = inter / (bw * bh + aw * ah - inter + 1e-7)
        maskf = jnp.where(piou > pt, 1.0, 0.0) * validf
        hit = hit_cell * maskf

        possum = jnp.sum(hit, axis=0, keepdims=True)
        posf = jnp.where(possum > 0.0, 1.0, 0.0)
        tp = jnp.max(hit * (tvec + 1.0), axis=0, keepdims=True)
        w_last = hit * jnp.where((tvec + 1.0) == tp, 1.0, 0.0)
        coords = jax.lax.dot_general(
            lab[:, 1:5], w_last, (((0,), (0,)), ((), ())),
            preferred_element_type=jnp.float32)
        clst_raw = jax.lax.dot_general(
            onehot, hit, (((1,), (0,)), ((), ())),
            preferred_element_type=jnp.float32)
        clst = jnp.where(clst_raw > 0.0, 1.0, 0.0)

        px = (jax.nn.sigmoid(raw_ref[base + 0:base + 1, :]) * 1.05 - 0.025 + gx) * stride
        py = (jax.nn.sigmoid(raw_ref[base + 1:base + 2, :]) * 1.05 - 0.025 + gy) * stride
        pw = jnp.exp(raw_ref[base + 2:base + 3, :]) * aw
        ph = jnp.exp(raw_ref[base + 3:base + 4, :]) * ah
        conf = jax.nn.sigmoid(raw_ref[base + 4:base + 5, :])
        clsp = jax.nn.sigmoid(raw_ref[base + 5:base + 85, :])

        px1 = px - pw * 0.5
        px2 = px + pw * 0.5
        py1 = py - ph * 0.5
        py2 = py + ph * 0.5
        iw = jnp.maximum(jnp.minimum(px2, bx2) - jnp.maximum(px1, bx1), 0.0)
        ih = jnp.maximum(jnp.minimum(py2, by2) - jnp.maximum(py1, by1), 0.0)
        ai = iw * ih
        area_p = pw * ph
        iou = ai / (area_p + area_b - ai + 1e-7)
        ioum = jnp.max(jnp.where(validf > 0.0, iou, -jnp.inf),
                       axis=0, keepdims=True)
        negf = jnp.where(ioum < nt, 1.0, 0.0) * has_box

        tx = coords[0:1]
        ty = coords[1:2]
        tw = coords[2:3]
        th = coords[3:4]
        tx1 = tx - tw * 0.5
        tx2 = tx + tw * 0.5
        ty1 = ty - th * 0.5
        ty2 = ty + th * 0.5
        iw2 = jnp.maximum(jnp.minimum(px2, tx2) - jnp.maximum(px1, tx1), 0.0)
        ih2 = jnp.maximum(jnp.minimum(py2, ty2) - jnp.maximum(py1, ty1), 0.0)
        ai2 = iw2 * ih2
        iou2 = ai2 / (area_p + tw * th - ai2 + 1e-7)
        ow = jnp.maximum(jnp.maximum(px2, tx2) - jnp.minimum(px1, tx1), 0.0)
        oh = jnp.maximum(jnp.maximum(py2, ty2) - jnp.minimum(py1, ty1), 0.0)
        r2 = (px - tx) * (px - tx) + (py - ty) * (py - ty)
        c2 = ow * ow + oh * oh
        diou = iou2 - r2 / (c2 + 1e-7)

        bce = -(posf * _slog(conf) + (1.0 - posf) * _slog(1.0 - conf))
        bcec = -(clst * _slog(clsp) + (1.0 - clst) * _slog(1.0 - clsp))

        s_pos = s_pos + jnp.sum(posf)
        s_neg = s_neg + jnp.sum(negf)
        s_box = s_box + jnp.sum((1.0 - diou) * posf)
        s_bp = s_bp + jnp.sum(bce * posf)
        s_bn = s_bn + jnp.sum(bce * negf)
        s_cls = s_cls + jnp.sum(bcec * posf)

    oidx = jax.lax.broadcasted_iota(jnp.int32, (1, 8), 1).astype(jnp.float32)
    vals = jnp.where(oidx == 0.0, s_pos,
           jnp.where(oidx == 1.0, s_neg,
           jnp.where(oidx == 2.0, s_box,
           jnp.where(oidx == 3.0, s_bp,
           jnp.where(oidx == 4.0, s_bn,
           jnp.where(oidx == 5.0, s_cls, 0.0))))))
    out_ref[...] = vals


def _run_scale(raw, label, pt, nt, size):
    batch = raw.shape[0]
    ss = size * size
    raw2 = raw.reshape(batch, 255, ss)
    out = pl.pallas_call(
        functools.partial(_scale_kernel, size=size),
        grid=(batch,),
        in_specs=[
            pl.BlockSpec((None, 255, ss), lambda b: (b, 0, 0)),
            pl.BlockSpec((None, _NBOX, 5), lambda b: (b, 0, 0)),
            pl.BlockSpec((1, 1), lambda b: (0, 0)),
            pl.BlockSpec((1, 1), lambda b: (0, 0)),
        ],
        out_specs=pl.BlockSpec((None, 1, 8), lambda b: (b, 0, 0)),
        out_shape=jax.ShapeDtypeStruct((batch, 1, 8), jnp.float32),
    )(raw2, label, pt.reshape(1, 1), nt.reshape(1, 1))
    return jnp.sum(out.reshape(batch, 8), axis=0)


def kernel(predict_76, predict_38, predict_19, label, pos_thresh=0.2,
           neg_thresh=0.7):
    pt = jnp.asarray(pos_thresh, jnp.float32)
    nt = jnp.asarray(neg_thresh, jnp.float32)
    loss_box = jnp.zeros((), jnp.float32)
    loss_obj = jnp.zeros((), jnp.float32)
    loss_cls = jnp.zeros((), jnp.float32)
    for raw in (predict_76, predict_38, predict_19):
        size = raw.shape[2]
        batch = raw.shape[0]
        sums = _run_scale(raw, label, pt, nt, size)
        pos_cnt = sums[0]
        neg_cnt = sums[1]
        use = (pos_cnt > 0.0) & (neg_cnt > 0.0)
        lb = sums[2] / pos_cnt / batch
        lo = sums[3] / pos_cnt / batch + sums[4] / neg_cnt / batch
        lc = sums[5] / (pos_cnt * 80.0) / batch
        loss_box = loss_box + jnp.where(use, lb, 0.0)
        loss_obj = loss_obj + jnp.where(use, lo, 0.0)
        loss_cls = loss_cls + jnp.where(use, lc, 0.0)
    loss = loss_box + loss_obj + loss_cls
    return (loss, loss_box, loss_obj, loss_cls)
